# Initial kernel scaffold; baseline (speedup 1.0000x reference)
import jax
import jax.numpy as jnp
from jax.experimental import pallas as pl


def kernel(ids, x, Wm, bm, emb_table0, emb_table1, norm_table, Wp):
    raise NotImplementedError("write your pallas kernel here")


# speedup vs baseline: 16.2180x; 16.2180x over previous
import numpy as np
import jax
import jax.numpy as jnp
from jax.experimental import pallas as pl
from jax.experimental.pallas import tpu as pltpu

_D_IN, _D_OUT = 128, 64
_LSH = [(16, 64), (32, 32)]
_NORM_BINS = 64
_NORM_THRESHOLD = 0.1
_TB = 512


def _lsh_consts_np(num_proj, num_bins, seed):
    rng = np.random.RandomState(seed)
    pm = rng.randn(_D_IN, num_proj).astype(np.float32)
    pm = pm / np.linalg.norm(pm, axis=0, keepdims=True)
    res = 2.0 / num_bins
    grid = (np.linspace(-1.0, 1.0, num_bins + 1)[:-1] + 0.5 * res).astype(np.float32)
    return pm, grid


_PM0, _GRID0 = _lsh_consts_np(16, 64, 100)
_PM1, _GRID1 = _lsh_consts_np(32, 32, 101)

_W0 = 16 * 65
_W1 = 32 * 33
_WLSH = _W0 + _W1
_WPAD = 2176
_PMT = np.zeros((_D_IN, _WPAD), dtype=np.float32)
_PMT[:, :_W0] = np.repeat(_PM0, 65, axis=1)
_PMT[:, _W0:_WLSH] = np.repeat(_PM1, 33, axis=1)
_THRESH = np.full((1, _WPAD), 1e30, dtype=np.float32)
_THRESH[0, :_W0] = np.tile(np.concatenate([[-1e30], _GRID0]).astype(np.float32), 16)
_THRESH[0, _W0:_WLSH] = np.tile(np.concatenate([[-1e30], _GRID1]).astype(np.float32), 32)
_NTHRESH = np.full((1, _NORM_BINS), -1e30, dtype=np.float32)
_NTHRESH[0, 1:] = np.linspace(0.0, 1.0, _NORM_BINS + 1)[1:-1].astype(np.float32)

_SEGMASK0 = (np.arange(_W0) % 65 != 0).astype(np.float32)[:, None]
_SEGMASK1 = (np.arange(_W1) % 33 != 0).astype(np.float32)[:, None]
_SEGMASKN = (np.arange(_NORM_BINS) != 0).astype(np.float32)[:, None]


def _tower_kernel(ids_ref, x_ref, wcat_ref, bm_ref, thresh_ref, dtab_ref,
                  nthresh_ref, dtn_ref, wp_ref, emb_ref, prod_ref, mask_ref):
    x = x_ref[...]
    n2 = jnp.sum(x * x, axis=1, keepdims=True)
    nrm = jnp.sqrt(n2)
    xn = x / jnp.maximum(nrm, 1e-12)
    hi = jax.lax.Precision.HIGHEST
    y = jnp.dot(xn, wcat_ref[...], preferred_element_type=jnp.float32)
    ge = (y[:, :_WPAD] > thresh_ref[...]).astype(jnp.float32)
    lsh = jnp.dot(ge, dtab_ref[...], preferred_element_type=jnp.float32,
                  precision=hi)
    emb = y[:, _WPAD:] + bm_ref[...]
    gn = (nrm > nthresh_ref[...]).astype(jnp.float32)
    hist = jnp.dot(gn, dtn_ref[...], preferred_element_type=jnp.float32,
                   precision=hi)
    e = emb + lsh + hist
    m = jnp.logical_or(nrm < _NORM_THRESHOLD, ids_ref[...] == 0)
    e = jnp.where(m, 0.0, e)
    emb_ref[...] = e
    prod_ref[...] = jnp.dot(e, wp_ref[...], preferred_element_type=jnp.float32,
                            precision=hi)
    mask_ref[...] = m


def kernel(ids, x, Wm, bm, emb_table0, emb_table1, norm_table, Wp):
    B, S = ids.shape
    N = B * S
    G = N // _TB
    x2 = x.reshape(N, _D_IN)
    ids2 = ids.reshape(N, 1)

    wcat = jnp.concatenate([jnp.asarray(_PMT), Wm], axis=1)
    dt0 = emb_table0 - jnp.roll(emb_table0, 1, axis=0) * jnp.asarray(_SEGMASK0)
    dt1 = emb_table1 - jnp.roll(emb_table1, 1, axis=0) * jnp.asarray(_SEGMASK1)
    dtab = jnp.concatenate(
        [dt0, dt1, jnp.zeros((_WPAD - _WLSH, _D_OUT), jnp.float32)], axis=0)
    dtn = norm_table - jnp.roll(norm_table, 1, axis=0) * jnp.asarray(_SEGMASKN)
    bm2 = bm.reshape(1, _D_OUT)

    full = lambda shape: pl.BlockSpec(shape, lambda i: (0, 0))
    emb2, prod2, mask2 = pl.pallas_call(
        _tower_kernel,
        grid=(G,),
        in_specs=[
            pl.BlockSpec((_TB, 1), lambda i: (i, 0)),
            pl.BlockSpec((_TB, _D_IN), lambda i: (i, 0)),
            full(wcat.shape),
            full((1, _D_OUT)),
            full((1, _WPAD)),
            full(dtab.shape),
            full((1, _NORM_BINS)),
            full(dtn.shape),
            full(Wp.shape),
        ],
        out_specs=[
            pl.BlockSpec((_TB, _D_OUT), lambda i: (i, 0)),
            pl.BlockSpec((_TB, _D_OUT), lambda i: (i, 0)),
            pl.BlockSpec((_TB, 1), lambda i: (i, 0)),
        ],
        out_shape=[
            jax.ShapeDtypeStruct((N, _D_OUT), jnp.float32),
            jax.ShapeDtypeStruct((N, _D_OUT), jnp.float32),
            jax.ShapeDtypeStruct((N, 1), jnp.bool_),
        ],
        compiler_params=pltpu.CompilerParams(
            dimension_semantics=("parallel",)),
    )(ids2, x2, wcat, bm2, jnp.asarray(_THRESH), dtab,
      jnp.asarray(_NTHRESH), dtn, Wp)

    return emb2.reshape(B, S, _D_OUT), prod2.reshape(B, S, _D_OUT), mask2.reshape(B, S)


# speedup vs baseline: 16.9533x; 1.0453x over previous
import numpy as np
import jax
import jax.numpy as jnp
from jax.experimental import pallas as pl
from jax.experimental.pallas import tpu as pltpu

_D_IN, _D_OUT = 128, 64
_LSH = [(16, 64), (32, 32)]
_NORM_BINS = 64
_NORM_THRESHOLD = 0.1
_TB = 1024


def _lsh_consts_np(num_proj, num_bins, seed):
    rng = np.random.RandomState(seed)
    pm = rng.randn(_D_IN, num_proj).astype(np.float32)
    pm = pm / np.linalg.norm(pm, axis=0, keepdims=True)
    res = 2.0 / num_bins
    grid = (np.linspace(-1.0, 1.0, num_bins + 1)[:-1] + 0.5 * res).astype(np.float32)
    return pm, grid


_PM0, _GRID0 = _lsh_consts_np(16, 64, 100)
_PM1, _GRID1 = _lsh_consts_np(32, 32, 101)

_W0 = 16 * 65
_W1 = 32 * 33
_WLSH = _W0 + _W1
_WPAD = 2176
_PMT = np.zeros((_D_IN, _WPAD), dtype=np.float32)
_PMT[:, :_W0] = np.repeat(_PM0, 65, axis=1)
_PMT[:, _W0:_WLSH] = np.repeat(_PM1, 33, axis=1)
_THRESH = np.full((1, _WPAD), 1e30, dtype=np.float32)
_THRESH[0, :_W0] = np.tile(np.concatenate([[-1e30], _GRID0]).astype(np.float32), 16)
_THRESH[0, _W0:_WLSH] = np.tile(np.concatenate([[-1e30], _GRID1]).astype(np.float32), 32)
_NTHRESH = np.full((1, _NORM_BINS), -1e30, dtype=np.float32)
_NTHRESH[0, 1:] = np.linspace(0.0, 1.0, _NORM_BINS + 1)[1:-1].astype(np.float32)

_SEGMASK0 = (np.arange(_W0) % 65 != 0).astype(np.float32)[:, None]
_SEGMASK1 = (np.arange(_W1) % 33 != 0).astype(np.float32)[:, None]
_SEGMASKN = (np.arange(_NORM_BINS) != 0).astype(np.float32)[:, None]


def _tower_kernel(ids_ref, x_ref, wcat_ref, bm_ref, thresh_ref, dtab_ref,
                  nthresh_ref, dtn_ref, wp_ref, emb_ref, prod_ref, mask_ref):
    x = x_ref[...]
    n2 = jnp.sum(x * x, axis=1, keepdims=True)
    nrm = jnp.sqrt(n2)
    xn = x / jnp.maximum(nrm, 1e-12)
    hi = jax.lax.Precision.HIGHEST
    y = jnp.dot(xn, wcat_ref[...], preferred_element_type=jnp.float32)
    ge = (y[:, :_WPAD] > thresh_ref[...]).astype(jnp.float32)
    lsh = jnp.dot(ge, dtab_ref[...], preferred_element_type=jnp.float32,
                  precision=hi)
    emb = y[:, _WPAD:] + bm_ref[...]
    gn = (nrm > nthresh_ref[...]).astype(jnp.float32)
    hist = jnp.dot(gn, dtn_ref[...], preferred_element_type=jnp.float32,
                   precision=hi)
    e = emb + lsh + hist
    m = jnp.logical_or(nrm < _NORM_THRESHOLD, ids_ref[...] == 0)
    e = jnp.where(m, 0.0, e)
    emb_ref[...] = e
    prod_ref[...] = jnp.dot(e, wp_ref[...], preferred_element_type=jnp.float32,
                            precision=hi)
    mask_ref[...] = m


def kernel(ids, x, Wm, bm, emb_table0, emb_table1, norm_table, Wp):
    B, S = ids.shape
    N = B * S
    G = N // _TB
    x2 = x.reshape(N, _D_IN)
    ids2 = ids.reshape(N, 1)

    wcat = jnp.concatenate([jnp.asarray(_PMT), Wm], axis=1)
    dt0 = emb_table0 - jnp.roll(emb_table0, 1, axis=0) * jnp.asarray(_SEGMASK0)
    dt1 = emb_table1 - jnp.roll(emb_table1, 1, axis=0) * jnp.asarray(_SEGMASK1)
    dtab = jnp.concatenate(
        [dt0, dt1, jnp.zeros((_WPAD - _WLSH, _D_OUT), jnp.float32)], axis=0)
    dtn = norm_table - jnp.roll(norm_table, 1, axis=0) * jnp.asarray(_SEGMASKN)
    bm2 = bm.reshape(1, _D_OUT)

    full = lambda shape: pl.BlockSpec(shape, lambda i: (0, 0))
    emb2, prod2, mask2 = pl.pallas_call(
        _tower_kernel,
        grid=(G,),
        in_specs=[
            pl.BlockSpec((_TB, 1), lambda i: (i, 0)),
            pl.BlockSpec((_TB, _D_IN), lambda i: (i, 0)),
            full(wcat.shape),
            full((1, _D_OUT)),
            full((1, _WPAD)),
            full(dtab.shape),
            full((1, _NORM_BINS)),
            full(dtn.shape),
            full(Wp.shape),
        ],
        out_specs=[
            pl.BlockSpec((_TB, _D_OUT), lambda i: (i, 0)),
            pl.BlockSpec((_TB, _D_OUT), lambda i: (i, 0)),
            pl.BlockSpec((_TB, 1), lambda i: (i, 0)),
        ],
        out_shape=[
            jax.ShapeDtypeStruct((N, _D_OUT), jnp.float32),
            jax.ShapeDtypeStruct((N, _D_OUT), jnp.float32),
            jax.ShapeDtypeStruct((N, 1), jnp.bool_),
        ],
        compiler_params=pltpu.CompilerParams(
            dimension_semantics=("parallel",)),
    )(ids2, x2, wcat, bm2, jnp.asarray(_THRESH), dtab,
      jnp.asarray(_NTHRESH), dtn, Wp)

    return emb2.reshape(B, S, _D_OUT), prod2.reshape(B, S, _D_OUT), mask2.reshape(B, S)
